# baseline (speedup 1.0000x reference)
import functools

import jax
import jax.numpy as jnp
from jax import lax
from jax.experimental import pallas as pl
from jax.experimental.pallas import tpu as pltpu
from jax.experimental.pallas import tpu_sc as plsc

LANES = 16
GROUPS = 2


def _build(B, T, D):
    info = plsc.get_sparse_core_info()
    NC, NS = info.num_cores, info.num_subcores
    NW = NC * NS
    t_per_w = T // NW
    b_per_g = B // GROUPS
    vregs_per_row = D // LANES

    mesh = plsc.VectorSubcoreMesh(core_axis_name="c", subcore_axis_name="s")

    @functools.partial(
        pl.kernel,
        mesh=mesh,
        out_type=jax.ShapeDtypeStruct((B, T, D), jnp.float32),
        scratch_types=[
            pltpu.VMEM((B * t_per_w,), jnp.int32),
            pltpu.VMEM((B * t_per_w, D), jnp.float32),
            pltpu.VMEM((t_per_w, D), jnp.float32),
        ]
        + [pltpu.SemaphoreType.DMA] * GROUPS
        + [pltpu.SemaphoreType.DMA, pltpu.SemaphoreType.DMA],
    )
    def emb(x_hbm, table_hbm, pos_hbm, out_hbm, idx_v, rows_v, pos_v, *sems):
        gsems, osem, isem = sems[:GROUPS], sems[GROUPS], sems[GROUPS + 1]
        wid = lax.axis_index("s") * NC + lax.axis_index("c")
        col = wid * t_per_w

        idx_cps = [
            pltpu.async_copy(
                x_hbm.at[b, pl.ds(col, t_per_w)],
                idx_v.at[pl.ds(b * t_per_w, t_per_w)],
                isem,
            )
            for b in range(B)
        ]
        gathers = [[] for _ in range(GROUPS)]
        for b in range(B):
            idx_cps[b].wait()
            gathers[b // b_per_g].append(
                pltpu.async_copy(
                    table_hbm.at[idx_v.at[pl.ds(b * t_per_w, t_per_w)]],
                    rows_v.at[pl.ds(b * t_per_w, t_per_w)],
                    gsems[b // b_per_g],
                )
            )
        pos_cp = pltpu.async_copy(pos_hbm.at[pl.ds(col, t_per_w)], pos_v, isem)

        outs = []
        for g in range(GROUPS):
            for cp in gathers[g]:
                cp.wait()
            if g == 0:
                pos_cp.wait()

            def row_body(r, carry, g=g):
                pos_regs = [
                    pos_v[r, pl.ds(j * LANES, LANES)] for j in range(vregs_per_row)
                ]
                for b in range(g * b_per_g, (g + 1) * b_per_g):
                    base = b * t_per_w
                    for j in range(vregs_per_row):
                        s = pl.ds(j * LANES, LANES)
                        rows_v[base + r, s] = rows_v[base + r, s] + pos_regs[j]
                return carry

            lax.fori_loop(0, t_per_w, row_body, 0)
            outs.extend(
                pltpu.async_copy(
                    rows_v.at[pl.ds(b * t_per_w, t_per_w)],
                    out_hbm.at[b, pl.ds(col, t_per_w)],
                    osem,
                )
                for b in range(g * b_per_g, (g + 1) * b_per_g)
            )
        for cp in outs:
            cp.wait()

    return emb


def kernel(x, token_table, pos_table):
    B, T = x.shape
    D = token_table.shape[1]
    return _build(B, T, D)(x.astype(jnp.int32), token_table, pos_table)
